# baseline (speedup 1.0000x reference)
import functools

import jax
import jax.numpy as jnp
from jax import lax
from jax.experimental import pallas as pl
from jax.experimental.pallas import tpu as pltpu
from jax.experimental.pallas import tpu_sc as plsc

NC = 2
NS = 16
NW = NC * NS
L = 16


def _detile_table(weights_t):
    D, V = weights_t.shape
    assert D == 32
    CW = 512
    NCOLS = V // CW
    TAIL = V - NCOLS * CW
    PER_W = (NCOLS + NW - 1) // NW

    mesh = plsc.VectorSubcoreMesh(
        core_axis_name="c", subcore_axis_name="s",
        num_cores=NC, num_subcores=NS)

    @functools.partial(
        pl.kernel,
        out_type=jax.ShapeDtypeStruct((V * D,), jnp.float32),
        mesh=mesh,
        compiler_params=pltpu.CompilerParams(
            use_tc_tiling_on_sc=True, needs_layout_passes=False),
        scratch_types=[
            pltpu.VMEM((D, CW), jnp.float32),
            pltpu.VMEM((D, CW), jnp.float32),
            pltpu.VMEM((CW * D,), jnp.float32),
            pltpu.VMEM((CW * D,), jnp.float32),
            pltpu.SemaphoreType.DMA,
            pltpu.SemaphoreType.DMA,
            pltpu.SemaphoreType.DMA,
            pltpu.SemaphoreType.DMA,
        ],
    )
    def detile(wt_hbm, out_hbm, g0, g1, t0, t1, gs0, gs1, os0, os1):
        wid = lax.axis_index("s") * NC + lax.axis_index("c")
        c_lo = wid * PER_W
        c_hi = jnp.minimum(c_lo + PER_W, NCOLS)

        gbuf = (g0, g1)
        tbuf = (t0, t1)
        gsem = (gs0, gs1)
        osem = (os0, os1)
        lane = lax.iota(jnp.int32, L)

        def gcopy(c, p):
            return pltpu.make_async_copy(
                wt_hbm.at[:, pl.ds(c * CW, CW)], gbuf[p], gsem[p])

        def ocopy(c, p):
            return pltpu.make_async_copy(
                tbuf[p], out_hbm.at[pl.ds(c * (CW * D), CW * D)], osem[p])

        def transpose(g, t):
            @plsc.parallel_loop(0, CW, 1, unroll=2)
            def body_v(v):
                vsplat = jnp.full((L,), v, jnp.int32)
                for half in range(2):
                    dvec = lane + (half * L)
                    vec = plsc.load_gather(g, [dvec, vsplat])
                    t[pl.ds(v * D + half * L, L)] = vec

        n = c_hi - c_lo

        @pl.when(n > 0)
        def _():
            gcopy(c_lo, 0).start()

        def body_c(i, _):
            c = c_lo + i
            for p in range(2):
                @pl.when(lax.rem(i, 2) == p)
                def _():
                    gcopy(c, p).wait()

                    @pl.when(i + 1 < n)
                    def _():
                        gcopy(c + 1, 1 - p).start()

                    @pl.when(i >= 2)
                    def _():
                        ocopy(c - 2, p).wait()
                    transpose(gbuf[p], tbuf[p])
                    ocopy(c, p).start()
            return 0

        lax.fori_loop(0, n, body_c, 0)

        for q in range(2):
            @pl.when((n >= 2) & (lax.rem(n - 2, 2) == q))
            def _(q=q):
                ocopy(c_hi - 2, q).wait()

            @pl.when((n >= 1) & (lax.rem(n - 1, 2) == q))
            def _(q=q):
                ocopy(c_hi - 1, q).wait()

    return detile(weights_t)


def kernel(token_ids, weights):
    B0, H = token_ids.shape
    V, D = weights.shape
    assert B0 % NW == 0 and D == 32 and H % 2 == 0
    W = B0 // NW
    KB = W // L
    JT = W // 128

    mesh = plsc.VectorSubcoreMesh(
        core_axis_name="c", subcore_axis_name="s",
        num_cores=NC, num_subcores=NS)

    @functools.partial(
        pl.kernel,
        out_type=jax.ShapeDtypeStruct((H, D // 8, B0 // 128, 8, 128),
                                      jnp.float32),
        mesh=mesh,
        compiler_params=pltpu.CompilerParams(
            use_tc_tiling_on_sc=False, needs_layout_passes=False),
        scratch_types=[
            pltpu.VMEM((H, W), jnp.int32),
            pltpu.VMEM((W, D), jnp.float32),
            pltpu.VMEM((W, D), jnp.float32),
            pltpu.VMEM((D // 8, JT, 8, 128), jnp.float32),
            pltpu.VMEM((D // 8, JT, 8, 128), jnp.float32),
            pltpu.SemaphoreType.DMA,
            pltpu.SemaphoreType.DMA,
            pltpu.SemaphoreType.DMA,
            pltpu.SemaphoreType.DMA,
            pltpu.SemaphoreType.DMA,
        ],
    )
    def emb(tids_hbm, table_hbm, out_hbm, idx_v, g0, g1, t0, t1,
            is_, gs0, gs1, os0, os1):
        wid = lax.axis_index("s") * NC + lax.axis_index("c")
        b0 = wid * W
        jb = wid * JT
        pltpu.async_copy(tids_hbm.at[:, pl.ds(b0, W)], idx_v, is_).wait()

        gbuf = (g0, g1)
        tbuf = (t0, t1)
        gsem = (gs0, gs1)
        osem = (os0, os1)

        def gstart(h, p):
            return pltpu.async_copy(
                table_hbm.at[idx_v.at[h]], gbuf[p], gsem[p])

        def ostart(h, p):
            return pltpu.async_copy(
                tbuf[p], out_hbm.at[h, :, pl.ds(jb, JT)], osem[p])

        def gwait(h, p):
            pltpu.make_async_copy(
                table_hbm.at[idx_v.at[h]], gbuf[p], gsem[p]).wait()

        def owait(h, p):
            pltpu.make_async_copy(
                tbuf[p], out_hbm.at[h, :, pl.ds(jb, JT)], osem[p]).wait()

        lane = lax.iota(jnp.int32, L)

        def transpose(g, t):
            @plsc.parallel_loop(0, D, 1, unroll=2)
            def body_d(d):
                dsplat = jnp.full((L,), d, jnp.int32)
                g8, r8 = d // 8, d % 8
                for k in range(KB):
                    bvec = lane + (k * L)
                    vec = plsc.load_gather(g, [bvec, dsplat])
                    t[g8, k // 8, r8, pl.ds((k % 8) * L, L)] = vec

        gstart(0, 0)
        gstart(1, 1)

        def body_h2(h2, _):
            for p in range(2):
                h = h2 * 2 + p
                gwait(h, p)

                @pl.when(h2 > 0)
                def _():
                    owait(h - 2, p)

                transpose(gbuf[p], tbuf[p])
                ostart(h, p)

                @pl.when(h2 < (H // 2 - 1))
                def _():
                    gstart(h + 2, p)
            return 0

        lax.fori_loop(0, H // 2, body_h2, 0)
        owait(H - 2, 0)
        owait(H - 1, 1)

    tids_t = jnp.swapaxes(token_ids, 0, 1).astype(jnp.int32)
    w_flat = _detile_table(jnp.swapaxes(weights, 0, 1))
    ntail = V % 512
    if ntail:
        tail = weights[V - ntail:, :].reshape(-1)
        w_flat = jax.lax.dynamic_update_slice(w_flat, tail, ((V - ntail) * D,))
    w_lin = w_flat.reshape(V, D)
    out5 = emb(tids_t, w_lin)
    return jnp.transpose(out5, (2, 4, 0, 1, 3)).reshape(B0, H, D)


# speedup vs baseline: 3.4932x; 3.4932x over previous
import functools

import jax
import jax.numpy as jnp
from jax import lax
from jax.experimental import pallas as pl
from jax.experimental.pallas import tpu as pltpu
from jax.experimental.pallas import tpu_sc as plsc

NC = 2
NS = 16
NW = NC * NS
L = 16


def _detile_table(weights_t):
    D, V = weights_t.shape
    assert D == 32
    CW = 512
    NCOLS = V // CW
    TAIL = V - NCOLS * CW
    PER_W = (NCOLS + NW - 1) // NW

    mesh = plsc.VectorSubcoreMesh(
        core_axis_name="c", subcore_axis_name="s",
        num_cores=NC, num_subcores=NS)

    @functools.partial(
        pl.kernel,
        out_type=jax.ShapeDtypeStruct((V * D,), jnp.float32),
        mesh=mesh,
        compiler_params=pltpu.CompilerParams(
            use_tc_tiling_on_sc=True, needs_layout_passes=False),
        scratch_types=[
            pltpu.VMEM((D, CW), jnp.float32),
            pltpu.VMEM((D, CW), jnp.float32),
            pltpu.VMEM((CW * D,), jnp.float32),
            pltpu.VMEM((CW * D,), jnp.float32),
            pltpu.SemaphoreType.DMA,
            pltpu.SemaphoreType.DMA,
            pltpu.SemaphoreType.DMA,
            pltpu.SemaphoreType.DMA,
        ],
    )
    def detile(wt_hbm, out_hbm, g0, g1, t0, t1, gs0, gs1, os0, os1):
        wid = lax.axis_index("s") * NC + lax.axis_index("c")
        c_lo = wid * PER_W
        c_hi = jnp.minimum(c_lo + PER_W, NCOLS)

        gbuf = (g0, g1)
        tbuf = (t0, t1)
        gsem = (gs0, gs1)
        osem = (os0, os1)
        lane = lax.iota(jnp.int32, L)

        def gcopy(c, p):
            return pltpu.make_async_copy(
                wt_hbm.at[:, pl.ds(c * CW, CW)], gbuf[p], gsem[p])

        def ocopy(c, p):
            return pltpu.make_async_copy(
                tbuf[p], out_hbm.at[pl.ds(c * (CW * D), CW * D)], osem[p])

        lane32 = lane * D

        def transpose(g, t):
            @plsc.parallel_loop(0, D, 1, unroll=2)
            def body_j(j):
                dvec = (lane + j) & (D - 1)
                for v0 in range(CW // L):
                    vvec = lane + (v0 * L)
                    vec = plsc.load_gather(g, [dvec, vvec])
                    sidx = (lane32 + (v0 * L * D)) + dvec
                    plsc.store_scatter(t, [sidx], vec)

        n = c_hi - c_lo

        @pl.when(n > 0)
        def _():
            gcopy(c_lo, 0).start()

        def body_c(i, _):
            c = c_lo + i
            for p in range(2):
                @pl.when(lax.rem(i, 2) == p)
                def _():
                    gcopy(c, p).wait()

                    @pl.when(i + 1 < n)
                    def _():
                        gcopy(c + 1, 1 - p).start()

                    @pl.when(i >= 2)
                    def _():
                        ocopy(c - 2, p).wait()
                    transpose(gbuf[p], tbuf[p])
                    ocopy(c, p).start()
            return 0

        lax.fori_loop(0, n, body_c, 0)

        for q in range(2):
            @pl.when((n >= 2) & (lax.rem(n - 2, 2) == q))
            def _(q=q):
                ocopy(c_hi - 2, q).wait()

            @pl.when((n >= 1) & (lax.rem(n - 1, 2) == q))
            def _(q=q):
                ocopy(c_hi - 1, q).wait()

    return detile(weights_t)


def kernel(token_ids, weights):
    B0, H = token_ids.shape
    V, D = weights.shape
    assert B0 % NW == 0 and D == 32 and H % 2 == 0
    W = B0 // NW
    KB = W // L
    JT = W // 128

    mesh = plsc.VectorSubcoreMesh(
        core_axis_name="c", subcore_axis_name="s",
        num_cores=NC, num_subcores=NS)

    @functools.partial(
        pl.kernel,
        out_type=jax.ShapeDtypeStruct((H, D * B0), jnp.float32),
        mesh=mesh,
        compiler_params=pltpu.CompilerParams(
            use_tc_tiling_on_sc=False, needs_layout_passes=False),
        scratch_types=[
            pltpu.VMEM((H, W), jnp.int32),
            pltpu.VMEM((W, D), jnp.float32),
            pltpu.VMEM((W, D), jnp.float32),
            pltpu.VMEM((D * W,), jnp.float32),
            pltpu.VMEM((D * W,), jnp.float32),
            pltpu.SemaphoreType.DMA,
            pltpu.SemaphoreType.DMA,
            pltpu.SemaphoreType.DMA,
            pltpu.SemaphoreType.DMA,
            pltpu.SemaphoreType.DMA,
        ],
    )
    def emb(tids_hbm, table_hbm, out_hbm, idx_v, g0, g1, t0, t1,
            is_, gs0, gs1, os0, os1):
        wid = lax.axis_index("s") * NC + lax.axis_index("c")
        b0 = wid * W
        jb = wid * JT
        pltpu.async_copy(tids_hbm.at[:, pl.ds(b0, W)], idx_v, is_).wait()

        gbuf = (g0, g1)
        tbuf = (t0, t1)
        gsem = (gs0, gs1)
        osem = (os0, os1)

        BT = B0 // 128 * 1024

        def gstart(h, p):
            return pltpu.async_copy(
                table_hbm.at[idx_v.at[h]], gbuf[p], gsem[p])

        def gwait(h, p):
            pltpu.make_async_copy(
                table_hbm.at[idx_v.at[h]], gbuf[p], gsem[p]).wait()

        def _ocopies(h, p):
            for d1 in range(D // 8):
                yield pltpu.make_async_copy(
                    tbuf[p].at[pl.ds(d1 * (JT * 1024), JT * 1024)],
                    out_hbm.at[h, pl.ds(d1 * BT + jb * 1024, JT * 1024)],
                    osem[p])

        def ostart(h, p):
            for c in _ocopies(h, p):
                c.start()

        def owait(h, p):
            for c in _ocopies(h, p):
                c.wait()

        lane = lax.iota(jnp.int32, L)

        def transpose(g, t):
            @plsc.parallel_loop(0, D, 1, unroll=2)
            def body_j(j):
                dvec = (lane + j) & (D - 1)
                dterm = ((dvec >> 3) << 12) | ((dvec & 7) << 7)
                sbase = dterm + lane
                for k in range(KB):
                    bvec = lane + (k * L)
                    vec = plsc.load_gather(g, [bvec, dvec])
                    sidx = sbase + ((k // 8) * 1024 + (k % 8) * L)
                    plsc.store_scatter(t, [sidx], vec)

        gstart(0, 0)
        gstart(1, 1)

        def body_h2(h2, _):
            for p in range(2):
                h = h2 * 2 + p
                gwait(h, p)

                @pl.when(h2 > 0)
                def _():
                    owait(h - 2, p)

                transpose(gbuf[p], tbuf[p])
                ostart(h, p)

                @pl.when(h2 < (H // 2 - 1))
                def _():
                    gstart(h + 2, p)
            return 0

        lax.fori_loop(0, H // 2, body_h2, 0)
        owait(H - 2, 0)
        owait(H - 1, 1)

    tids_t = jnp.swapaxes(token_ids, 0, 1).astype(jnp.int32)
    w_flat = _detile_table(jnp.swapaxes(weights, 0, 1))
    ntail = V % 512
    if ntail:
        tail = weights[V - ntail:, :].reshape(-1)
        w_flat = jax.lax.dynamic_update_slice(w_flat, tail, ((V - ntail) * D,))
    w_lin = w_flat.reshape(V, D)
    out2 = emb(tids_t, w_lin)
    out5 = out2.reshape(H, D // 8, B0 // 128, 8, 128)
    return jnp.transpose(out5, (2, 4, 0, 1, 3)).reshape(B0, H, D)
